# baseline (speedup 1.0000x reference)
import functools

import jax
import jax.numpy as jnp
import numpy as np
from jax.experimental import pallas as pl
from jax.experimental.pallas import tpu as pltpu

_N = 10000
_E = 160000
_G = 64
_K = 8000
_NC = _E // _K


def _leaky(v):
    return jnp.where(v > 0, v, 0.2 * v)


def _gat_layer_body(h_ref, src_ref, dst_ref, W_ref, As_ref, Ad_ref, B_ref,
                    bias_ref, g_ref, be_ref, p0_ref, p1_ref, p2_ref, p3_ref,
                    out_ref, xp_s, nb_s, acc_s, *, H, HC, mode):
    p = pl.program_id(0)
    j = pl.program_id(1)
    RC = 20
    CH = _N // RC

    @pl.when((p == 0) & (j == 0))
    def _init_phase0():
        for r in range(RC):
            sl = pl.ds(r * CH, CH)
            xp_c = jnp.dot(h_ref[sl, :], W_ref[:, :],
                           preferred_element_type=jnp.float32)
            xp_s[sl, :] = xp_c
            asn_c = jnp.dot(xp_c, As_ref[:, :], preferred_element_type=jnp.float32)
            adn_c = jnp.dot(xp_c, Ad_ref[:, :], preferred_element_type=jnp.float32)
            nb_s[sl, 0:H] = asn_c
            nb_s[sl, 32:32 + H] = adn_c
            nb_s[sl, 64:64 + H] = _leaky(asn_c + adn_c)

    @pl.when(p == 0)
    def _phase0_edges():
        def body(i, carry):
            s = src_ref[0, 0, i]
            d = dst_ref[0, 0, i]
            row_s = nb_s[pl.ds(s, 1), :]
            row_d = nb_s[pl.ds(d, 1), :]
            alpha = _leaky(row_s[:, 0:H] + row_d[:, 32:32 + H])
            nb_s[pl.ds(d, 1), 64:64 + H] = jnp.maximum(row_d[:, 64:64 + H], alpha)
            return carry
        jax.lax.fori_loop(0, _K, body, 0)

    @pl.when((p == 1) & (j == 0))
    def _init_phase1():
        for r in range(RC):
            sl = pl.ds(r * CH, CH)
            asn = nb_s[sl, 0:H]
            adn = nb_s[sl, 32:32 + H]
            es = jnp.exp(_leaky(asn + adn) - nb_s[sl, 64:64 + H])
            nb_s[sl, 96:96 + H] = es
            acc_s[sl, :] = xp_s[sl, :] * jnp.dot(
                es, B_ref[:, :], preferred_element_type=jnp.float32)

    @pl.when(p == 1)
    def _phase1_edges():
        def body(i, carry):
            s = src_ref[0, 0, i]
            d = dst_ref[0, 0, i]
            row_s = nb_s[pl.ds(s, 1), :]
            row_d = nb_s[pl.ds(d, 1), :]
            alpha = _leaky(row_s[:, 0:H] + row_d[:, 32:32 + H])
            ea = jnp.exp(alpha - row_d[:, 64:64 + H])
            nb_s[pl.ds(d, 1), 96:96 + H] = row_d[:, 96:96 + H] + ea
            contrib = xp_s[pl.ds(s, 1), :] * jnp.dot(
                ea, B_ref[:, :], preferred_element_type=jnp.float32)
            acc_s[pl.ds(d, 1), :] = acc_s[pl.ds(d, 1), :] + contrib
            return carry
        jax.lax.fori_loop(0, _K, body, 0)

    @pl.when((p == 1) & (j == _NC - 1))
    def _epilogue():
        ssum = jnp.zeros((1, HC), jnp.float32)
        ssq = jnp.zeros((1, HC), jnp.float32)
        for r in range(RC):
            sl = pl.ds(r * CH, CH)
            den = nb_s[sl, 96:96 + H] + 1e-16
            dex = jnp.dot(den, B_ref[:, :], preferred_element_type=jnp.float32)
            v = acc_s[sl, :] / dex + bias_ref[0:1, :]
            acc_s[sl, :] = v
            ssum = ssum + jnp.sum(v, axis=0, keepdims=True)
            ssq = ssq + jnp.sum(v * v, axis=0, keepdims=True)
        m = ssum / _N
        var = ssq / _N - m * m
        scale = g_ref[0:1, :] * jax.lax.rsqrt(var + 1e-5)
        shift = be_ref[0:1, :] - m * scale
        if mode < 3:
            s2 = jnp.zeros((1, out_ref.shape[1]), jnp.float32)
            q2 = jnp.zeros((1, out_ref.shape[1]), jnp.float32)
            for r in range(RC):
                sl = pl.ds(r * CH, CH)
                v = jnp.maximum(acc_s[sl, :] * scale + shift, 0.0)
                r2 = jnp.dot(v, p0_ref[:, :],
                             preferred_element_type=jnp.float32) + p1_ref[0:1, :]
                out_ref[sl, :] = r2
                s2 = s2 + jnp.sum(r2, axis=0, keepdims=True)
                q2 = q2 + jnp.sum(r2 * r2, axis=0, keepdims=True)
            m2 = s2 / _N
            var2 = q2 / _N - m2 * m2
            sc2 = p2_ref[0:1, :] * jax.lax.rsqrt(var2 + 1e-5)
            sh2 = p3_ref[0:1, :] - m2 * sc2
            for r in range(RC):
                sl = pl.ds(r * CH, CH)
                out_ref[sl, :] = jnp.maximum(out_ref[sl, :] * sc2 + sh2, 0.0)
        else:
            pooled = jnp.zeros((_G, HC), jnp.float32)
            cnt = jnp.zeros((_G, 1), jnp.float32)
            for r in range(RC):
                sl = pl.ds(r * CH, CH)
                v = jnp.maximum(acc_s[sl, :] * scale + shift, 0.0)
                gids = jax.lax.broadcasted_iota(jnp.int32, (_G, CH), 0)
                mask = (p0_ref[0:1, r * CH:(r + 1) * CH] == gids).astype(jnp.float32)
                pooled = pooled + jnp.dot(mask, v,
                                          preferred_element_type=jnp.float32)
                cnt = cnt + jnp.sum(mask, axis=1, keepdims=True)
            pooled = pooled / jnp.maximum(cnt, 1.0)
            logits = jnp.dot(pooled, p1_ref[:, :],
                             preferred_element_type=jnp.float32)
            out_ref[:, :] = logits + p2_ref[0:1, :]


def _gat_layer(h, src2d, dst2d, W, att_src, att_dst, bias, g, be,
               extras, *, mode, out_shape):
    H, C = att_src.shape
    HC = H * C
    B = np.zeros((H, HC), np.float32)
    for hh in range(H):
        B[hh, hh * C:(hh + 1) * C] = 1.0
    As_j = jnp.reshape(att_src, (HC, 1)) * jnp.asarray(B).T
    Ad_j = jnp.reshape(att_dst, (HC, 1)) * jnp.asarray(B).T
    B_j = jnp.asarray(B)
    bias2 = bias.reshape(1, -1)
    g2 = g.reshape(1, -1)
    be2 = be.reshape(1, -1)

    body = functools.partial(_gat_layer_body, H=H, HC=HC, mode=mode)

    def full(a):
        return pl.BlockSpec(a.shape, lambda p, j: (0,) * a.ndim)

    p0, p1, p2, p3 = extras
    in_specs = [
        full(h),
        pl.BlockSpec((1, 1, _K), lambda p, j: (j, 0, 0), memory_space=pltpu.SMEM),
        pl.BlockSpec((1, 1, _K), lambda p, j: (j, 0, 0), memory_space=pltpu.SMEM),
        full(W), full(As_j), full(Ad_j), full(B_j),
        full(bias2), full(g2), full(be2),
        full(p0), full(p1), full(p2), full(p3),
    ]
    out = pl.pallas_call(
        body,
        grid=(2, _NC),
        in_specs=in_specs,
        out_specs=pl.BlockSpec(out_shape, lambda p, j: (0, 0)),
        out_shape=jax.ShapeDtypeStruct(out_shape, jnp.float32),
        scratch_shapes=[
            pltpu.VMEM((_N, HC), jnp.float32),
            pltpu.VMEM((_N, 128), jnp.float32),
            pltpu.VMEM((_N, HC), jnp.float32),
        ],
        compiler_params=pltpu.CompilerParams(
            vmem_limit_bytes=110 * 1024 * 1024),
    )(h, src2d, dst2d, W, As_j, Ad_j, B_j, bias2, g2, be2, p0, p1, p2, p3)
    return out


def kernel(x, edge_index, batch, W1, as1, ad1, b1, g1, be1, Wl1, bl1, gl1, bel1,
           W2, as2, ad2, b2, g2, be2, Wl2, bl2, gl2, bel2,
           W3, as3, ad3, b3, g3, be3, Wout, bout):
    src2d = edge_index[0].reshape(_NC, 1, _K)
    dst2d = edge_index[1].reshape(_NC, 1, _K)

    h = _gat_layer(
        x, src2d, dst2d, W1, as1, ad1, b1, g1, be1,
        (Wl1, bl1.reshape(1, -1), gl1.reshape(1, -1), bel1.reshape(1, -1)),
        mode=1, out_shape=(_N, 16))
    h = _gat_layer(
        h, src2d, dst2d, W2, as2, ad2, b2, g2, be2,
        (Wl2, bl2.reshape(1, -1), gl2.reshape(1, -1), bel2.reshape(1, -1)),
        mode=2, out_shape=(_N, 32))
    out = _gat_layer(
        h, src2d, dst2d, W3, as3, ad3, b3, g3, be3,
        (batch.reshape(1, -1), Wout, bout.reshape(1, -1),
         jnp.zeros((1, 1), jnp.float32)),
        mode=3, out_shape=(_G, 10))
    return out


# speedup vs baseline: 1.1607x; 1.1607x over previous
import functools

import jax
import jax.numpy as jnp
import numpy as np
from jax.experimental import pallas as pl
from jax.experimental.pallas import tpu as pltpu

_N = 10000
_E = 160000
_G = 64
_K = 8000
_NC = _E // _K


def _leaky(v):
    return jnp.where(v > 0, v, 0.2 * v)


def _gat_layer_body(h_ref, src_ref, dst_ref, W_ref, As_ref, Ad_ref, B_ref,
                    bias_ref, g_ref, be_ref, p0_ref, p1_ref, p2_ref, p3_ref,
                    out_ref, xp_s, nb_s, acc_s, *, H, HC, mode):
    p = pl.program_id(0)
    j = pl.program_id(1)
    RC = 20
    CH = _N // RC

    @pl.when((p == 0) & (j == 0))
    def _init_phase0():
        for r in range(RC):
            sl = pl.ds(r * CH, CH)
            xp_c = jnp.dot(h_ref[sl, :], W_ref[:, :],
                           preferred_element_type=jnp.float32)
            xp_s[sl, :] = xp_c
            asn_c = jnp.dot(xp_c, As_ref[:, :], preferred_element_type=jnp.float32)
            adn_c = jnp.dot(xp_c, Ad_ref[:, :], preferred_element_type=jnp.float32)
            nb_s[sl, 0:H] = asn_c
            nb_s[sl, 32:32 + H] = adn_c
            nb_s[sl, 64:64 + H] = _leaky(asn_c + adn_c)

    @pl.when(p == 0)
    def _phase0_edges():
        def body(i, carry):
            s = src_ref[0, 0, i]
            d = dst_ref[0, 0, i]
            row_s = nb_s[pl.ds(s, 1), :]
            row_d = nb_s[pl.ds(d, 1), :]
            alpha = _leaky(row_s[:, 0:H] + row_d[:, 32:32 + H])
            nb_s[pl.ds(d, 1), 64:64 + H] = jnp.maximum(row_d[:, 64:64 + H], alpha)
            return carry
        jax.lax.fori_loop(0, _K, body, 0, unroll=8)

    @pl.when((p == 1) & (j == 0))
    def _init_phase1():
        for r in range(RC):
            sl = pl.ds(r * CH, CH)
            asn = nb_s[sl, 0:H]
            adn = nb_s[sl, 32:32 + H]
            es = jnp.exp(_leaky(asn + adn) - nb_s[sl, 64:64 + H])
            nb_s[sl, 96:96 + H] = es
            acc_s[sl, :] = xp_s[sl, :] * jnp.dot(
                es, B_ref[:, :], preferred_element_type=jnp.float32)

    @pl.when(p == 1)
    def _phase1_edges():
        def body(i, carry):
            s = src_ref[0, 0, i]
            d = dst_ref[0, 0, i]
            row_s = nb_s[pl.ds(s, 1), :]
            row_d = nb_s[pl.ds(d, 1), :]
            alpha = _leaky(row_s[:, 0:H] + row_d[:, 32:32 + H])
            ea = jnp.exp(alpha - row_d[:, 64:64 + H])
            nb_s[pl.ds(d, 1), 96:96 + H] = row_d[:, 96:96 + H] + ea
            contrib = xp_s[pl.ds(s, 1), :] * jnp.dot(
                ea, B_ref[:, :], preferred_element_type=jnp.float32)
            acc_s[pl.ds(d, 1), :] = acc_s[pl.ds(d, 1), :] + contrib
            return carry
        jax.lax.fori_loop(0, _K, body, 0, unroll=8)

    @pl.when((p == 1) & (j == _NC - 1))
    def _epilogue():
        ssum = jnp.zeros((1, HC), jnp.float32)
        ssq = jnp.zeros((1, HC), jnp.float32)
        for r in range(RC):
            sl = pl.ds(r * CH, CH)
            den = nb_s[sl, 96:96 + H] + 1e-16
            dex = jnp.dot(den, B_ref[:, :], preferred_element_type=jnp.float32)
            v = acc_s[sl, :] / dex + bias_ref[0:1, :]
            acc_s[sl, :] = v
            ssum = ssum + jnp.sum(v, axis=0, keepdims=True)
            ssq = ssq + jnp.sum(v * v, axis=0, keepdims=True)
        m = ssum / _N
        var = ssq / _N - m * m
        scale = g_ref[0:1, :] * jax.lax.rsqrt(var + 1e-5)
        shift = be_ref[0:1, :] - m * scale
        if mode < 3:
            s2 = jnp.zeros((1, out_ref.shape[1]), jnp.float32)
            q2 = jnp.zeros((1, out_ref.shape[1]), jnp.float32)
            for r in range(RC):
                sl = pl.ds(r * CH, CH)
                v = jnp.maximum(acc_s[sl, :] * scale + shift, 0.0)
                r2 = jnp.dot(v, p0_ref[:, :],
                             preferred_element_type=jnp.float32) + p1_ref[0:1, :]
                out_ref[sl, :] = r2
                s2 = s2 + jnp.sum(r2, axis=0, keepdims=True)
                q2 = q2 + jnp.sum(r2 * r2, axis=0, keepdims=True)
            m2 = s2 / _N
            var2 = q2 / _N - m2 * m2
            sc2 = p2_ref[0:1, :] * jax.lax.rsqrt(var2 + 1e-5)
            sh2 = p3_ref[0:1, :] - m2 * sc2
            for r in range(RC):
                sl = pl.ds(r * CH, CH)
                out_ref[sl, :] = jnp.maximum(out_ref[sl, :] * sc2 + sh2, 0.0)
        else:
            pooled = jnp.zeros((_G, HC), jnp.float32)
            cnt = jnp.zeros((_G, 1), jnp.float32)
            for r in range(RC):
                sl = pl.ds(r * CH, CH)
                v = jnp.maximum(acc_s[sl, :] * scale + shift, 0.0)
                gids = jax.lax.broadcasted_iota(jnp.int32, (_G, CH), 0)
                mask = (p0_ref[0:1, r * CH:(r + 1) * CH] == gids).astype(jnp.float32)
                pooled = pooled + jnp.dot(mask, v,
                                          preferred_element_type=jnp.float32)
                cnt = cnt + jnp.sum(mask, axis=1, keepdims=True)
            pooled = pooled / jnp.maximum(cnt, 1.0)
            logits = jnp.dot(pooled, p1_ref[:, :],
                             preferred_element_type=jnp.float32)
            out_ref[:, :] = logits + p2_ref[0:1, :]


def _gat_layer(h, src2d, dst2d, W, att_src, att_dst, bias, g, be,
               extras, *, mode, out_shape):
    H, C = att_src.shape
    HC = H * C
    B = np.zeros((H, HC), np.float32)
    for hh in range(H):
        B[hh, hh * C:(hh + 1) * C] = 1.0
    As_j = jnp.reshape(att_src, (HC, 1)) * jnp.asarray(B).T
    Ad_j = jnp.reshape(att_dst, (HC, 1)) * jnp.asarray(B).T
    B_j = jnp.asarray(B)
    bias2 = bias.reshape(1, -1)
    g2 = g.reshape(1, -1)
    be2 = be.reshape(1, -1)

    body = functools.partial(_gat_layer_body, H=H, HC=HC, mode=mode)

    def full(a):
        return pl.BlockSpec(a.shape, lambda p, j: (0,) * a.ndim)

    p0, p1, p2, p3 = extras
    in_specs = [
        full(h),
        pl.BlockSpec((1, 1, _K), lambda p, j: (j, 0, 0), memory_space=pltpu.SMEM),
        pl.BlockSpec((1, 1, _K), lambda p, j: (j, 0, 0), memory_space=pltpu.SMEM),
        full(W), full(As_j), full(Ad_j), full(B_j),
        full(bias2), full(g2), full(be2),
        full(p0), full(p1), full(p2), full(p3),
    ]
    out = pl.pallas_call(
        body,
        grid=(2, _NC),
        in_specs=in_specs,
        out_specs=pl.BlockSpec(out_shape, lambda p, j: (0, 0)),
        out_shape=jax.ShapeDtypeStruct(out_shape, jnp.float32),
        scratch_shapes=[
            pltpu.VMEM((_N, HC), jnp.float32),
            pltpu.VMEM((_N, 128), jnp.float32),
            pltpu.VMEM((_N, HC), jnp.float32),
        ],
        compiler_params=pltpu.CompilerParams(
            vmem_limit_bytes=110 * 1024 * 1024),
    )(h, src2d, dst2d, W, As_j, Ad_j, B_j, bias2, g2, be2, p0, p1, p2, p3)
    return out


def kernel(x, edge_index, batch, W1, as1, ad1, b1, g1, be1, Wl1, bl1, gl1, bel1,
           W2, as2, ad2, b2, g2, be2, Wl2, bl2, gl2, bel2,
           W3, as3, ad3, b3, g3, be3, Wout, bout):
    src2d = edge_index[0].reshape(_NC, 1, _K)
    dst2d = edge_index[1].reshape(_NC, 1, _K)

    h = _gat_layer(
        x, src2d, dst2d, W1, as1, ad1, b1, g1, be1,
        (Wl1, bl1.reshape(1, -1), gl1.reshape(1, -1), bel1.reshape(1, -1)),
        mode=1, out_shape=(_N, 16))
    h = _gat_layer(
        h, src2d, dst2d, W2, as2, ad2, b2, g2, be2,
        (Wl2, bl2.reshape(1, -1), gl2.reshape(1, -1), bel2.reshape(1, -1)),
        mode=2, out_shape=(_N, 32))
    out = _gat_layer(
        h, src2d, dst2d, W3, as3, ad3, b3, g3, be3,
        (batch.reshape(1, -1), Wout, bout.reshape(1, -1),
         jnp.zeros((1, 1), jnp.float32)),
        mode=3, out_shape=(_G, 10))
    return out
